# baseline (speedup 1.0000x reference)
import functools
import jax
import jax.numpy as jnp
from jax import lax
from jax.experimental import pallas as pl
from jax.experimental.pallas import tpu as pltpu
from jax.experimental.pallas import tpu_sc as plsc

N_NODES = 10000
FEAT = 128
NC = 2
NS = 16
NW = NC * NS
LANE = 128
ECHUNK = 64
N_PAD = 10240
ROWS_PER_TILE = N_PAD // NS
_SPLIT0 = 0.28


def _mesh():
    return plsc.VectorSubcoreMesh(core_axis_name="c", subcore_axis_name="s")


def _make_sc_degree(ch):
    @functools.partial(
        pl.kernel,
        out_type=jax.ShapeDtypeStruct((NC, N_PAD, FEAT), jnp.float32),
        mesh=_mesh(),
        scratch_types=[
            pltpu.VMEM((ch, LANE), jnp.int32),
            pltpu.VMEM((LANE, FEAT), jnp.float32),
            pltpu.VMEM_SHARED((N_PAD, FEAT), jnp.float32),
            pltpu.SemaphoreType.DMA,
        ],
    )
    def deg_kernel(dst_hbm, zeros_hbm, ones_hbm, out_hbm, dst_v, ones_v, acc, sem):
        c = lax.axis_index("c")
        s = lax.axis_index("s")
        wid = s * NC + c
        base = s * ROWS_PER_TILE
        pltpu.sync_copy(zeros_hbm, acc.at[pl.ds(base, ROWS_PER_TILE)])
        pltpu.sync_copy(ones_hbm, ones_v)
        pltpu.sync_copy(dst_hbm.at[wid], dst_v)
        plsc.subcore_barrier()

        def body(j, carry):
            pltpu.sync_copy(ones_v, acc.at[dst_v.at[j]], add=True)
            return carry

        lax.fori_loop(0, ch, body, 0)
        plsc.subcore_barrier()
        pltpu.sync_copy(
            acc.at[pl.ds(base, ROWS_PER_TILE)],
            out_hbm.at[c].at[pl.ds(base, ROWS_PER_TILE)],
        )

    return deg_kernel


def _make_sc_edge_scatter(ch_max, ch0, ch1):
    @functools.partial(
        pl.kernel,
        out_type=jax.ShapeDtypeStruct((NC, N_PAD, FEAT), jnp.float32),
        mesh=_mesh(),
        scratch_types=[
            pltpu.VMEM((ch_max, LANE), jnp.int32),
            pltpu.VMEM((ch_max, LANE), jnp.int32),
            pltpu.VMEM((LANE, FEAT), jnp.float32),
            pltpu.VMEM_SHARED((N_PAD, FEAT), jnp.float32),
            pltpu.SemaphoreType.DMA,
        ],
    )
    def edge_kernel(ms_hbm, src_hbm, dst_hbm, zeros_hbm, out_hbm,
                    src_v, dst_v, rows_v, acc, sem):
        c = lax.axis_index("c")
        s = lax.axis_index("s")
        wid = c * NS + s
        base = s * ROWS_PER_TILE
        nch = jnp.where(c == 0, ch0, ch1)
        pltpu.sync_copy(zeros_hbm, acc.at[pl.ds(base, ROWS_PER_TILE)])
        pltpu.sync_copy(src_hbm.at[wid], src_v)
        pltpu.sync_copy(dst_hbm.at[wid], dst_v)
        plsc.subcore_barrier()

        def body(j, carry):
            pltpu.async_copy(ms_hbm.at[src_v.at[j]], rows_v, sem).wait()
            pltpu.sync_copy(rows_v, acc.at[dst_v.at[j]], add=True)
            return carry

        lax.fori_loop(0, nch, body, 0)
        plsc.subcore_barrier()
        pltpu.sync_copy(
            acc.at[pl.ds(base, ROWS_PER_TILE)],
            out_hbm.at[c].at[pl.ds(base, ROWS_PER_TILE)],
        )

    return edge_kernel


_BLK = 1024
_GRID = N_PAD // _BLK


def _dis_from_deg(degp0, degp1):
    deg = degp0[:, 0:1] + degp1[:, 0:1] + 1.0
    return lax.rsqrt(deg)


def _tc_ms1_body(deg_ref, x_ref, w_ref, out_ref):
    dis = _dis_from_deg(deg_ref[0], deg_ref[1])
    out_ref[...] = dis * jnp.dot(x_ref[...], w_ref[...],
                                 preferred_element_type=jnp.float32)


def _tc_mid_body(deg_ref, p_ref, ms_ref, w_ref, b_ref, out_ref):
    dis = _dis_from_deg(deg_ref[0], deg_ref[1])
    agg = p_ref[0] + p_ref[1] + ms_ref[...]
    h = jax.nn.relu(dis * agg + b_ref[...])
    out_ref[...] = dis * jnp.dot(h, w_ref[...],
                                 preferred_element_type=jnp.float32)


def _tc_head_body(deg_ref, q_ref, ms_ref, b_ref, wl_ref, bl_ref, out_ref):
    dis = _dis_from_deg(deg_ref[0], deg_ref[1])
    agg = q_ref[0] + q_ref[1] + ms_ref[...]
    h = jax.nn.relu(dis * agg + b_ref[...])
    out_ref[...] = jnp.sum(h * wl_ref[...], axis=1, keepdims=True) + bl_ref[...]


def _deg_spec():
    return pl.BlockSpec((NC, _BLK, FEAT), lambda i: (0, i, 0))


def _row_spec():
    return pl.BlockSpec((_BLK, FEAT), lambda i: (i, 0))


def _part_spec():
    return pl.BlockSpec((NC, _BLK, FEAT), lambda i: (0, i, 0))


def _full_spec(shape):
    nd = len(shape)
    return pl.BlockSpec(shape, lambda i: (0,) * nd)


def _tc_ms1(degp, x, w):
    return pl.pallas_call(
        _tc_ms1_body,
        grid=(_GRID,),
        in_specs=[_deg_spec(), _row_spec(), _full_spec((FEAT, FEAT))],
        out_specs=_row_spec(),
        out_shape=jax.ShapeDtypeStruct((N_PAD, FEAT), jnp.float32),
    )(degp, x, w)


def _tc_mid(degp, part, ms, w, b):
    return pl.pallas_call(
        _tc_mid_body,
        grid=(_GRID,),
        in_specs=[_deg_spec(), _part_spec(), _row_spec(),
                  _full_spec((FEAT, FEAT)), _full_spec((1, FEAT))],
        out_specs=_row_spec(),
        out_shape=jax.ShapeDtypeStruct((N_PAD, FEAT), jnp.float32),
    )(degp, part, ms, w, b)


def _tc_head(degp, part, ms, b, wl_row, bl):
    return pl.pallas_call(
        _tc_head_body,
        grid=(_GRID,),
        in_specs=[_deg_spec(), _part_spec(), _row_spec(),
                  _full_spec((1, FEAT)), _full_spec((1, FEAT)),
                  _full_spec((1, 1))],
        out_specs=pl.BlockSpec((_BLK, 1), lambda i: (i, 0)),
        out_shape=jax.ShapeDtypeStruct((N_PAD, 1), jnp.float32),
    )(degp, part, ms, b, wl_row, bl)


@jax.jit
def _run(x, edge_index, W1, b1, W2, b2, Wl, bl):
    n, _ = x.shape
    e = edge_index.shape[1]
    src = edge_index[0].astype(jnp.int32)
    dst = edge_index[1].astype(jnp.int32)

    ch = -(-e // (NW * LANE))
    e_pad = NW * ch * LANE
    pad_idx = jnp.full((e_pad - e,), N_PAD - 1, dtype=jnp.int32)
    dst_r = jnp.concatenate([dst, pad_idx]).reshape(NW, ch, LANE)

    n_chunks = -(-e // LANE)
    ch0 = int(n_chunks * _SPLIT0) // NS
    ch1 = -(-(n_chunks - NS * ch0) // NS)
    ch_max = max(ch0, ch1)
    e0 = NS * ch0 * LANE

    def _slabs(idx):
        a = jnp.full((NC, NS, ch_max, LANE), N_PAD - 1, dtype=jnp.int32)
        p0 = idx[:e0].reshape(NS, ch0, LANE)
        p1_flat = idx[e0:]
        p1 = jnp.concatenate(
            [p1_flat,
             jnp.full((NS * ch1 * LANE - p1_flat.shape[0],), N_PAD - 1,
                      dtype=jnp.int32)]).reshape(NS, ch1, LANE)
        a = a.at[0, :, :ch0].set(p0).at[1, :, :ch1].set(p1)
        return a.reshape(NW, ch_max, LANE)

    src_s = _slabs(src)
    dst_s = _slabs(dst)

    x_pad = jnp.zeros((N_PAD, FEAT), jnp.float32).at[:n].set(x)
    ones128 = jnp.ones((LANE, FEAT), jnp.float32)
    zeros128 = jnp.zeros((ROWS_PER_TILE, FEAT), jnp.float32)

    degp = _make_sc_degree(ch)(dst_r, zeros128, ones128)

    ms1 = _tc_ms1(degp, x_pad, W1)
    p1 = _make_sc_edge_scatter(ch_max, ch0, ch1)(ms1, src_s, dst_s, zeros128)
    ms2 = _tc_mid(degp, p1, ms1, W2, b1.reshape(1, FEAT))
    p2 = _make_sc_edge_scatter(ch_max, ch0, ch1)(ms2, src_s, dst_s, zeros128)
    out = _tc_head(degp, p2, ms2, b2.reshape(1, FEAT),
                   Wl.reshape(1, FEAT), bl.reshape(1, 1))
    return out[:n, 0]


def kernel(x, edge_index, W1, b1, W2, b2, Wl, bl):
    return _run(x, edge_index, W1, b1, W2, b2, Wl, bl)


# speedup vs baseline: 1.0745x; 1.0745x over previous
import functools
import jax
import jax.numpy as jnp
from jax import lax
from jax.experimental import pallas as pl
from jax.experimental.pallas import tpu as pltpu
from jax.experimental.pallas import tpu_sc as plsc

N_NODES = 10000
FEAT = 128
NC = 2
NS = 16
NW = NC * NS
LANE = 128
ECHUNK = 64
N_PAD = 10240
ROWS_PER_TILE = N_PAD // NS
_SPLIT0 = 0.40


def _mesh():
    return plsc.VectorSubcoreMesh(core_axis_name="c", subcore_axis_name="s")


def _make_sc_degree(ch):
    @functools.partial(
        pl.kernel,
        out_type=jax.ShapeDtypeStruct((NC, N_PAD, FEAT), jnp.float32),
        mesh=_mesh(),
        scratch_types=[
            pltpu.VMEM((ch, LANE), jnp.int32),
            pltpu.VMEM((LANE, FEAT), jnp.float32),
            pltpu.VMEM_SHARED((N_PAD, FEAT), jnp.float32),
            pltpu.SemaphoreType.DMA,
        ],
    )
    def deg_kernel(dst_hbm, zeros_hbm, ones_hbm, out_hbm, dst_v, ones_v, acc, sem):
        c = lax.axis_index("c")
        s = lax.axis_index("s")
        wid = s * NC + c
        base = s * ROWS_PER_TILE
        pltpu.sync_copy(zeros_hbm, acc.at[pl.ds(base, ROWS_PER_TILE)])
        pltpu.sync_copy(ones_hbm, ones_v)
        pltpu.sync_copy(dst_hbm.at[wid], dst_v)
        plsc.subcore_barrier()

        def body(j, carry):
            pltpu.sync_copy(ones_v, acc.at[dst_v.at[j]], add=True)
            return carry

        lax.fori_loop(0, ch, body, 0)
        plsc.subcore_barrier()
        pltpu.sync_copy(
            acc.at[pl.ds(base, ROWS_PER_TILE)],
            out_hbm.at[c].at[pl.ds(base, ROWS_PER_TILE)],
        )

    return deg_kernel


def _make_sc_edge_scatter(ch_max, ch0, ch1):
    @functools.partial(
        pl.kernel,
        out_type=jax.ShapeDtypeStruct((NC, N_PAD, FEAT), jnp.float32),
        mesh=_mesh(),
        scratch_types=[
            pltpu.VMEM((ch_max, LANE), jnp.int32),
            pltpu.VMEM((ch_max, LANE), jnp.int32),
            pltpu.VMEM((LANE, FEAT), jnp.float32),
            pltpu.VMEM_SHARED((N_PAD, FEAT), jnp.float32),
            pltpu.SemaphoreType.DMA,
        ],
    )
    def edge_kernel(ms_hbm, src_hbm, dst_hbm, zeros_hbm, out_hbm,
                    src_v, dst_v, rows_v, acc, sem):
        c = lax.axis_index("c")
        s = lax.axis_index("s")
        wid = c * NS + s
        base = s * ROWS_PER_TILE
        nch = jnp.where(c == 0, ch0, ch1)
        pltpu.sync_copy(zeros_hbm, acc.at[pl.ds(base, ROWS_PER_TILE)])
        pltpu.sync_copy(src_hbm.at[wid], src_v)
        pltpu.sync_copy(dst_hbm.at[wid], dst_v)
        plsc.subcore_barrier()

        def body(j, carry):
            pltpu.async_copy(ms_hbm.at[src_v.at[j]], rows_v, sem).wait()
            pltpu.sync_copy(rows_v, acc.at[dst_v.at[j]], add=True)
            return carry

        lax.fori_loop(0, nch, body, 0)
        plsc.subcore_barrier()
        pltpu.sync_copy(
            acc.at[pl.ds(base, ROWS_PER_TILE)],
            out_hbm.at[c].at[pl.ds(base, ROWS_PER_TILE)],
        )

    return edge_kernel


_BLK = 1024
_GRID = N_PAD // _BLK


def _dis_from_deg(degp0, degp1):
    deg = degp0[:, 0:1] + degp1[:, 0:1] + 1.0
    return lax.rsqrt(deg)


def _tc_ms1_body(deg_ref, x_ref, w_ref, out_ref):
    dis = _dis_from_deg(deg_ref[0], deg_ref[1])
    out_ref[...] = dis * jnp.dot(x_ref[...], w_ref[...],
                                 preferred_element_type=jnp.float32)


def _tc_mid_body(deg_ref, p_ref, ms_ref, w_ref, b_ref, out_ref):
    dis = _dis_from_deg(deg_ref[0], deg_ref[1])
    agg = p_ref[0] + p_ref[1] + ms_ref[...]
    h = jax.nn.relu(dis * agg + b_ref[...])
    out_ref[...] = dis * jnp.dot(h, w_ref[...],
                                 preferred_element_type=jnp.float32)


def _tc_head_body(deg_ref, q_ref, ms_ref, b_ref, wl_ref, bl_ref, out_ref):
    dis = _dis_from_deg(deg_ref[0], deg_ref[1])
    agg = q_ref[0] + q_ref[1] + ms_ref[...]
    h = jax.nn.relu(dis * agg + b_ref[...])
    out_ref[...] = jnp.sum(h * wl_ref[...], axis=1, keepdims=True) + bl_ref[...]


def _deg_spec():
    return pl.BlockSpec((NC, _BLK, FEAT), lambda i: (0, i, 0))


def _row_spec():
    return pl.BlockSpec((_BLK, FEAT), lambda i: (i, 0))


def _part_spec():
    return pl.BlockSpec((NC, _BLK, FEAT), lambda i: (0, i, 0))


def _full_spec(shape):
    nd = len(shape)
    return pl.BlockSpec(shape, lambda i: (0,) * nd)


def _tc_ms1(degp, x, w):
    return pl.pallas_call(
        _tc_ms1_body,
        grid=(_GRID,),
        in_specs=[_deg_spec(), _row_spec(), _full_spec((FEAT, FEAT))],
        out_specs=_row_spec(),
        out_shape=jax.ShapeDtypeStruct((N_PAD, FEAT), jnp.float32),
    )(degp, x, w)


def _tc_mid(degp, part, ms, w, b):
    return pl.pallas_call(
        _tc_mid_body,
        grid=(_GRID,),
        in_specs=[_deg_spec(), _part_spec(), _row_spec(),
                  _full_spec((FEAT, FEAT)), _full_spec((1, FEAT))],
        out_specs=_row_spec(),
        out_shape=jax.ShapeDtypeStruct((N_PAD, FEAT), jnp.float32),
    )(degp, part, ms, w, b)


def _tc_head(degp, part, ms, b, wl_row, bl):
    return pl.pallas_call(
        _tc_head_body,
        grid=(_GRID,),
        in_specs=[_deg_spec(), _part_spec(), _row_spec(),
                  _full_spec((1, FEAT)), _full_spec((1, FEAT)),
                  _full_spec((1, 1))],
        out_specs=pl.BlockSpec((_BLK, 1), lambda i: (i, 0)),
        out_shape=jax.ShapeDtypeStruct((N_PAD, 1), jnp.float32),
    )(degp, part, ms, b, wl_row, bl)


@jax.jit
def _run(x, edge_index, W1, b1, W2, b2, Wl, bl):
    n, _ = x.shape
    e = edge_index.shape[1]
    src = edge_index[0].astype(jnp.int32)
    dst = edge_index[1].astype(jnp.int32)

    ch = -(-e // (NW * LANE))
    e_pad = NW * ch * LANE
    pad_idx = jnp.full((e_pad - e,), N_PAD - 1, dtype=jnp.int32)
    dst_r = jnp.concatenate([dst, pad_idx]).reshape(NW, ch, LANE)

    n_chunks = -(-e // LANE)
    ch0 = int(n_chunks * _SPLIT0) // NS
    ch1 = -(-(n_chunks - NS * ch0) // NS)
    ch_max = max(ch0, ch1)
    e0 = NS * ch0 * LANE

    def _slabs(idx):
        a = jnp.full((NC, NS, ch_max, LANE), N_PAD - 1, dtype=jnp.int32)
        p0 = idx[:e0].reshape(NS, ch0, LANE)
        p1_flat = idx[e0:]
        p1 = jnp.concatenate(
            [p1_flat,
             jnp.full((NS * ch1 * LANE - p1_flat.shape[0],), N_PAD - 1,
                      dtype=jnp.int32)]).reshape(NS, ch1, LANE)
        a = a.at[0, :, :ch0].set(p0).at[1, :, :ch1].set(p1)
        return a.reshape(NW, ch_max, LANE)

    src_s = _slabs(src)
    dst_s = _slabs(dst)

    x_pad = jnp.zeros((N_PAD, FEAT), jnp.float32).at[:n].set(x)
    ones128 = jnp.ones((LANE, FEAT), jnp.float32)
    zeros128 = jnp.zeros((ROWS_PER_TILE, FEAT), jnp.float32)

    degp = _make_sc_degree(ch)(dst_r, zeros128, ones128)

    ms1 = _tc_ms1(degp, x_pad, W1)
    p1 = _make_sc_edge_scatter(ch_max, ch0, ch1)(ms1, src_s, dst_s, zeros128)
    ms2 = _tc_mid(degp, p1, ms1, W2, b1.reshape(1, FEAT))
    p2 = _make_sc_edge_scatter(ch_max, ch0, ch1)(ms2, src_s, dst_s, zeros128)
    out = _tc_head(degp, p2, ms2, b2.reshape(1, FEAT),
                   Wl.reshape(1, FEAT), bl.reshape(1, 1))
    return out[:n, 0]


def kernel(x, edge_index, W1, b1, W2, b2, Wl, bl):
    return _run(x, edge_index, W1, b1, W2, b2, Wl, bl)


# speedup vs baseline: 1.1103x; 1.0334x over previous
import functools
import jax
import jax.numpy as jnp
from jax import lax
from jax.experimental import pallas as pl
from jax.experimental.pallas import tpu as pltpu
from jax.experimental.pallas import tpu_sc as plsc

N_NODES = 10000
FEAT = 128
NC = 2
NS = 16
NW = NC * NS
LANE = 128
ECHUNK = 64
N_PAD = 10240
ROWS_PER_TILE = N_PAD // NS
_SPLIT0 = 0.44


def _mesh():
    return plsc.VectorSubcoreMesh(core_axis_name="c", subcore_axis_name="s")


def _make_sc_degree(ch):
    @functools.partial(
        pl.kernel,
        out_type=jax.ShapeDtypeStruct((NC, N_PAD, FEAT), jnp.float32),
        mesh=_mesh(),
        scratch_types=[
            pltpu.VMEM((ch, LANE), jnp.int32),
            pltpu.VMEM((LANE, FEAT), jnp.float32),
            pltpu.VMEM_SHARED((N_PAD, FEAT), jnp.float32),
            pltpu.SemaphoreType.DMA,
        ],
    )
    def deg_kernel(dst_hbm, zeros_hbm, ones_hbm, out_hbm, dst_v, ones_v, acc, sem):
        c = lax.axis_index("c")
        s = lax.axis_index("s")
        wid = s * NC + c
        base = s * ROWS_PER_TILE
        pltpu.sync_copy(zeros_hbm, acc.at[pl.ds(base, ROWS_PER_TILE)])
        pltpu.sync_copy(ones_hbm, ones_v)
        pltpu.sync_copy(dst_hbm.at[wid], dst_v)
        plsc.subcore_barrier()

        def body(j, carry):
            pltpu.sync_copy(ones_v, acc.at[dst_v.at[j]], add=True)
            return carry

        lax.fori_loop(0, ch, body, 0)
        plsc.subcore_barrier()
        pltpu.sync_copy(
            acc.at[pl.ds(base, ROWS_PER_TILE)],
            out_hbm.at[c].at[pl.ds(base, ROWS_PER_TILE)],
        )

    return deg_kernel


def _make_sc_edge_scatter(ch_max, ch0, ch1):
    @functools.partial(
        pl.kernel,
        out_type=jax.ShapeDtypeStruct((NC, N_PAD, FEAT), jnp.float32),
        mesh=_mesh(),
        scratch_types=[
            pltpu.VMEM((ch_max, LANE), jnp.int32),
            pltpu.VMEM((ch_max, LANE), jnp.int32),
            pltpu.VMEM((LANE, FEAT), jnp.float32),
            pltpu.VMEM_SHARED((N_PAD, FEAT), jnp.float32),
            pltpu.SemaphoreType.DMA,
        ],
    )
    def edge_kernel(ms_hbm, src_hbm, dst_hbm, zeros_hbm, out_hbm,
                    src_v, dst_v, rows_v, acc, sem):
        c = lax.axis_index("c")
        s = lax.axis_index("s")
        wid = c * NS + s
        base = s * ROWS_PER_TILE
        nch = jnp.where(c == 0, ch0, ch1)
        pltpu.sync_copy(zeros_hbm, acc.at[pl.ds(base, ROWS_PER_TILE)])
        pltpu.sync_copy(src_hbm.at[wid], src_v)
        pltpu.sync_copy(dst_hbm.at[wid], dst_v)
        plsc.subcore_barrier()

        def body(j, carry):
            pltpu.async_copy(ms_hbm.at[src_v.at[j]], rows_v, sem).wait()
            pltpu.sync_copy(rows_v, acc.at[dst_v.at[j]], add=True)
            return carry

        lax.fori_loop(0, nch, body, 0)
        plsc.subcore_barrier()
        pltpu.sync_copy(
            acc.at[pl.ds(base, ROWS_PER_TILE)],
            out_hbm.at[c].at[pl.ds(base, ROWS_PER_TILE)],
        )

    return edge_kernel


_BLK = 1024
_GRID = N_PAD // _BLK


def _dis_from_deg(degp0, degp1):
    deg = degp0[:, 0:1] + degp1[:, 0:1] + 1.0
    return lax.rsqrt(deg)


def _tc_ms1_body(deg_ref, x_ref, w_ref, out_ref):
    dis = _dis_from_deg(deg_ref[0], deg_ref[1])
    out_ref[...] = dis * jnp.dot(x_ref[...], w_ref[...],
                                 preferred_element_type=jnp.float32)


def _tc_mid_body(deg_ref, p_ref, ms_ref, w_ref, b_ref, out_ref):
    dis = _dis_from_deg(deg_ref[0], deg_ref[1])
    agg = p_ref[0] + p_ref[1] + ms_ref[...]
    h = jax.nn.relu(dis * agg + b_ref[...])
    out_ref[...] = dis * jnp.dot(h, w_ref[...],
                                 preferred_element_type=jnp.float32)


def _tc_head_body(deg_ref, q_ref, ms_ref, b_ref, wl_ref, bl_ref, out_ref):
    dis = _dis_from_deg(deg_ref[0], deg_ref[1])
    agg = q_ref[0] + q_ref[1] + ms_ref[...]
    h = jax.nn.relu(dis * agg + b_ref[...])
    out_ref[...] = jnp.sum(h * wl_ref[...], axis=1, keepdims=True) + bl_ref[...]


def _deg_spec():
    return pl.BlockSpec((NC, _BLK, FEAT), lambda i: (0, i, 0))


def _row_spec():
    return pl.BlockSpec((_BLK, FEAT), lambda i: (i, 0))


def _part_spec():
    return pl.BlockSpec((NC, _BLK, FEAT), lambda i: (0, i, 0))


def _full_spec(shape):
    nd = len(shape)
    return pl.BlockSpec(shape, lambda i: (0,) * nd)


def _tc_ms1(degp, x, w):
    return pl.pallas_call(
        _tc_ms1_body,
        grid=(_GRID,),
        in_specs=[_deg_spec(), _row_spec(), _full_spec((FEAT, FEAT))],
        out_specs=_row_spec(),
        out_shape=jax.ShapeDtypeStruct((N_PAD, FEAT), jnp.float32),
    )(degp, x, w)


def _tc_mid(degp, part, ms, w, b):
    return pl.pallas_call(
        _tc_mid_body,
        grid=(_GRID,),
        in_specs=[_deg_spec(), _part_spec(), _row_spec(),
                  _full_spec((FEAT, FEAT)), _full_spec((1, FEAT))],
        out_specs=_row_spec(),
        out_shape=jax.ShapeDtypeStruct((N_PAD, FEAT), jnp.float32),
    )(degp, part, ms, w, b)


def _tc_head(degp, part, ms, b, wl_row, bl):
    return pl.pallas_call(
        _tc_head_body,
        grid=(_GRID,),
        in_specs=[_deg_spec(), _part_spec(), _row_spec(),
                  _full_spec((1, FEAT)), _full_spec((1, FEAT)),
                  _full_spec((1, 1))],
        out_specs=pl.BlockSpec((_BLK, 1), lambda i: (i, 0)),
        out_shape=jax.ShapeDtypeStruct((N_PAD, 1), jnp.float32),
    )(degp, part, ms, b, wl_row, bl)


@jax.jit
def _run(x, edge_index, W1, b1, W2, b2, Wl, bl):
    n, _ = x.shape
    e = edge_index.shape[1]
    src = edge_index[0].astype(jnp.int32)
    dst = edge_index[1].astype(jnp.int32)

    ch = -(-e // (NW * LANE))
    e_pad = NW * ch * LANE
    pad_idx = jnp.full((e_pad - e,), N_PAD - 1, dtype=jnp.int32)
    dst_r = jnp.concatenate([dst, pad_idx]).reshape(NW, ch, LANE)

    n_chunks = -(-e // LANE)
    ch0 = int(n_chunks * _SPLIT0) // NS
    ch1 = -(-(n_chunks - NS * ch0) // NS)
    ch_max = max(ch0, ch1)
    e0 = NS * ch0 * LANE

    def _slabs(idx):
        a = jnp.full((NC, NS, ch_max, LANE), N_PAD - 1, dtype=jnp.int32)
        p0 = idx[:e0].reshape(NS, ch0, LANE)
        p1_flat = idx[e0:]
        p1 = jnp.concatenate(
            [p1_flat,
             jnp.full((NS * ch1 * LANE - p1_flat.shape[0],), N_PAD - 1,
                      dtype=jnp.int32)]).reshape(NS, ch1, LANE)
        a = a.at[0, :, :ch0].set(p0).at[1, :, :ch1].set(p1)
        return a.reshape(NW, ch_max, LANE)

    src_s = _slabs(src)
    dst_s = _slabs(dst)

    x_pad = jnp.zeros((N_PAD, FEAT), jnp.float32).at[:n].set(x)
    ones128 = jnp.ones((LANE, FEAT), jnp.float32)
    zeros128 = jnp.zeros((ROWS_PER_TILE, FEAT), jnp.float32)

    degp = _make_sc_degree(ch)(dst_r, zeros128, ones128)

    ms1 = _tc_ms1(degp, x_pad, W1)
    p1 = _make_sc_edge_scatter(ch_max, ch0, ch1)(ms1, src_s, dst_s, zeros128)
    ms2 = _tc_mid(degp, p1, ms1, W2, b1.reshape(1, FEAT))
    p2 = _make_sc_edge_scatter(ch_max, ch0, ch1)(ms2, src_s, dst_s, zeros128)
    out = _tc_head(degp, p2, ms2, b2.reshape(1, FEAT),
                   Wl.reshape(1, FEAT), bl.reshape(1, 1))
    return out[:n, 0]


def kernel(x, edge_index, W1, b1, W2, b2, Wl, bl):
    return _run(x, edge_index, W1, b1, W2, b2, Wl, bl)


# speedup vs baseline: 1.1550x; 1.0402x over previous
import functools
import jax
import jax.numpy as jnp
from jax import lax
from jax.experimental import pallas as pl
from jax.experimental.pallas import tpu as pltpu
from jax.experimental.pallas import tpu_sc as plsc

N_NODES = 10000
FEAT = 128
NC = 2
NS = 16
NW = NC * NS
LANE = 128
ECHUNK = 64
N_PAD = 10240
ROWS_PER_TILE = N_PAD // NS
_SPLIT0 = 0.48


def _mesh():
    return plsc.VectorSubcoreMesh(core_axis_name="c", subcore_axis_name="s")


def _make_sc_degree(ch):
    @functools.partial(
        pl.kernel,
        out_type=jax.ShapeDtypeStruct((NC, N_PAD, FEAT), jnp.float32),
        mesh=_mesh(),
        scratch_types=[
            pltpu.VMEM((ch, LANE), jnp.int32),
            pltpu.VMEM((LANE, FEAT), jnp.float32),
            pltpu.VMEM_SHARED((N_PAD, FEAT), jnp.float32),
            pltpu.SemaphoreType.DMA,
        ],
    )
    def deg_kernel(dst_hbm, zeros_hbm, ones_hbm, out_hbm, dst_v, ones_v, acc, sem):
        c = lax.axis_index("c")
        s = lax.axis_index("s")
        wid = s * NC + c
        base = s * ROWS_PER_TILE
        pltpu.sync_copy(zeros_hbm, acc.at[pl.ds(base, ROWS_PER_TILE)])
        pltpu.sync_copy(ones_hbm, ones_v)
        pltpu.sync_copy(dst_hbm.at[wid], dst_v)
        plsc.subcore_barrier()

        def body(j, carry):
            pltpu.sync_copy(ones_v, acc.at[dst_v.at[j]], add=True)
            return carry

        lax.fori_loop(0, ch, body, 0)
        plsc.subcore_barrier()
        pltpu.sync_copy(
            acc.at[pl.ds(base, ROWS_PER_TILE)],
            out_hbm.at[c].at[pl.ds(base, ROWS_PER_TILE)],
        )

    return deg_kernel


def _make_sc_edge_scatter(ch_max, ch0, ch1):
    @functools.partial(
        pl.kernel,
        out_type=jax.ShapeDtypeStruct((NC, N_PAD, FEAT), jnp.float32),
        mesh=_mesh(),
        scratch_types=[
            pltpu.VMEM((ch_max, LANE), jnp.int32),
            pltpu.VMEM((ch_max, LANE), jnp.int32),
            pltpu.VMEM((LANE, FEAT), jnp.float32),
            pltpu.VMEM_SHARED((N_PAD, FEAT), jnp.float32),
            pltpu.SemaphoreType.DMA,
        ],
    )
    def edge_kernel(ms_hbm, src_hbm, dst_hbm, zeros_hbm, out_hbm,
                    src_v, dst_v, rows_v, acc, sem):
        c = lax.axis_index("c")
        s = lax.axis_index("s")
        wid = c * NS + s
        base = s * ROWS_PER_TILE
        nch = jnp.where(c == 0, ch0, ch1)
        pltpu.sync_copy(zeros_hbm, acc.at[pl.ds(base, ROWS_PER_TILE)])
        pltpu.sync_copy(src_hbm.at[wid], src_v)
        pltpu.sync_copy(dst_hbm.at[wid], dst_v)
        plsc.subcore_barrier()

        def body(j, carry):
            pltpu.async_copy(ms_hbm.at[src_v.at[j]], rows_v, sem).wait()
            pltpu.sync_copy(rows_v, acc.at[dst_v.at[j]], add=True)
            return carry

        lax.fori_loop(0, nch, body, 0)
        plsc.subcore_barrier()
        pltpu.sync_copy(
            acc.at[pl.ds(base, ROWS_PER_TILE)],
            out_hbm.at[c].at[pl.ds(base, ROWS_PER_TILE)],
        )

    return edge_kernel


_BLK = 1024
_GRID = N_PAD // _BLK


def _dis_from_deg(degp0, degp1):
    deg = degp0[:, 0:1] + degp1[:, 0:1] + 1.0
    return lax.rsqrt(deg)


def _tc_ms1_body(deg_ref, x_ref, w_ref, out_ref):
    dis = _dis_from_deg(deg_ref[0], deg_ref[1])
    out_ref[...] = dis * jnp.dot(x_ref[...], w_ref[...],
                                 preferred_element_type=jnp.float32)


def _tc_mid_body(deg_ref, p_ref, ms_ref, w_ref, b_ref, out_ref):
    dis = _dis_from_deg(deg_ref[0], deg_ref[1])
    agg = p_ref[0] + p_ref[1] + ms_ref[...]
    h = jax.nn.relu(dis * agg + b_ref[...])
    out_ref[...] = dis * jnp.dot(h, w_ref[...],
                                 preferred_element_type=jnp.float32)


def _tc_head_body(deg_ref, q_ref, ms_ref, b_ref, wl_ref, bl_ref, out_ref):
    dis = _dis_from_deg(deg_ref[0], deg_ref[1])
    agg = q_ref[0] + q_ref[1] + ms_ref[...]
    h = jax.nn.relu(dis * agg + b_ref[...])
    out_ref[...] = jnp.sum(h * wl_ref[...], axis=1, keepdims=True) + bl_ref[...]


def _deg_spec():
    return pl.BlockSpec((NC, _BLK, FEAT), lambda i: (0, i, 0))


def _row_spec():
    return pl.BlockSpec((_BLK, FEAT), lambda i: (i, 0))


def _part_spec():
    return pl.BlockSpec((NC, _BLK, FEAT), lambda i: (0, i, 0))


def _full_spec(shape):
    nd = len(shape)
    return pl.BlockSpec(shape, lambda i: (0,) * nd)


def _tc_ms1(degp, x, w):
    return pl.pallas_call(
        _tc_ms1_body,
        grid=(_GRID,),
        in_specs=[_deg_spec(), _row_spec(), _full_spec((FEAT, FEAT))],
        out_specs=_row_spec(),
        out_shape=jax.ShapeDtypeStruct((N_PAD, FEAT), jnp.float32),
    )(degp, x, w)


def _tc_mid(degp, part, ms, w, b):
    return pl.pallas_call(
        _tc_mid_body,
        grid=(_GRID,),
        in_specs=[_deg_spec(), _part_spec(), _row_spec(),
                  _full_spec((FEAT, FEAT)), _full_spec((1, FEAT))],
        out_specs=_row_spec(),
        out_shape=jax.ShapeDtypeStruct((N_PAD, FEAT), jnp.float32),
    )(degp, part, ms, w, b)


def _tc_head(degp, part, ms, b, wl_row, bl):
    return pl.pallas_call(
        _tc_head_body,
        grid=(_GRID,),
        in_specs=[_deg_spec(), _part_spec(), _row_spec(),
                  _full_spec((1, FEAT)), _full_spec((1, FEAT)),
                  _full_spec((1, 1))],
        out_specs=pl.BlockSpec((_BLK, 1), lambda i: (i, 0)),
        out_shape=jax.ShapeDtypeStruct((N_PAD, 1), jnp.float32),
    )(degp, part, ms, b, wl_row, bl)


@jax.jit
def _run(x, edge_index, W1, b1, W2, b2, Wl, bl):
    n, _ = x.shape
    e = edge_index.shape[1]
    src = edge_index[0].astype(jnp.int32)
    dst = edge_index[1].astype(jnp.int32)

    ch = -(-e // (NW * LANE))
    e_pad = NW * ch * LANE
    pad_idx = jnp.full((e_pad - e,), N_PAD - 1, dtype=jnp.int32)
    dst_r = jnp.concatenate([dst, pad_idx]).reshape(NW, ch, LANE)

    n_chunks = -(-e // LANE)
    ch0 = int(n_chunks * _SPLIT0) // NS
    ch1 = -(-(n_chunks - NS * ch0) // NS)
    ch_max = max(ch0, ch1)
    e0 = NS * ch0 * LANE

    def _slabs(idx):
        a = jnp.full((NC, NS, ch_max, LANE), N_PAD - 1, dtype=jnp.int32)
        p0 = idx[:e0].reshape(NS, ch0, LANE)
        p1_flat = idx[e0:]
        p1 = jnp.concatenate(
            [p1_flat,
             jnp.full((NS * ch1 * LANE - p1_flat.shape[0],), N_PAD - 1,
                      dtype=jnp.int32)]).reshape(NS, ch1, LANE)
        a = a.at[0, :, :ch0].set(p0).at[1, :, :ch1].set(p1)
        return a.reshape(NW, ch_max, LANE)

    src_s = _slabs(src)
    dst_s = _slabs(dst)

    x_pad = jnp.zeros((N_PAD, FEAT), jnp.float32).at[:n].set(x)
    ones128 = jnp.ones((LANE, FEAT), jnp.float32)
    zeros128 = jnp.zeros((ROWS_PER_TILE, FEAT), jnp.float32)

    degp = _make_sc_degree(ch)(dst_r, zeros128, ones128)

    ms1 = _tc_ms1(degp, x_pad, W1)
    p1 = _make_sc_edge_scatter(ch_max, ch0, ch1)(ms1, src_s, dst_s, zeros128)
    ms2 = _tc_mid(degp, p1, ms1, W2, b1.reshape(1, FEAT))
    p2 = _make_sc_edge_scatter(ch_max, ch0, ch1)(ms2, src_s, dst_s, zeros128)
    out = _tc_head(degp, p2, ms2, b2.reshape(1, FEAT),
                   Wl.reshape(1, FEAT), bl.reshape(1, 1))
    return out[:n, 0]


def kernel(x, edge_index, W1, b1, W2, b2, Wl, bl):
    return _run(x, edge_index, W1, b1, W2, b2, Wl, bl)


# speedup vs baseline: 1.1786x; 1.0205x over previous
import functools
import jax
import jax.numpy as jnp
from jax import lax
from jax.experimental import pallas as pl
from jax.experimental.pallas import tpu as pltpu
from jax.experimental.pallas import tpu_sc as plsc

N_NODES = 10000
FEAT = 128
NC = 2
NS = 16
NW = NC * NS
LANE = 128
ECHUNK = 64
N_PAD = 10240
ROWS_PER_TILE = N_PAD // NS
_SPLIT0 = 0.50


def _mesh():
    return plsc.VectorSubcoreMesh(core_axis_name="c", subcore_axis_name="s")


def _make_sc_degree(ch):
    @functools.partial(
        pl.kernel,
        out_type=jax.ShapeDtypeStruct((NC, N_PAD, FEAT), jnp.float32),
        mesh=_mesh(),
        scratch_types=[
            pltpu.VMEM((ch, LANE), jnp.int32),
            pltpu.VMEM((LANE, FEAT), jnp.float32),
            pltpu.VMEM_SHARED((N_PAD, FEAT), jnp.float32),
            pltpu.SemaphoreType.DMA,
        ],
    )
    def deg_kernel(dst_hbm, zeros_hbm, ones_hbm, out_hbm, dst_v, ones_v, acc, sem):
        c = lax.axis_index("c")
        s = lax.axis_index("s")
        wid = s * NC + c
        base = s * ROWS_PER_TILE
        pltpu.sync_copy(zeros_hbm, acc.at[pl.ds(base, ROWS_PER_TILE)])
        pltpu.sync_copy(ones_hbm, ones_v)
        pltpu.sync_copy(dst_hbm.at[wid], dst_v)
        plsc.subcore_barrier()

        def body(j, carry):
            pltpu.sync_copy(ones_v, acc.at[dst_v.at[j]], add=True)
            return carry

        lax.fori_loop(0, ch, body, 0)
        plsc.subcore_barrier()
        pltpu.sync_copy(
            acc.at[pl.ds(base, ROWS_PER_TILE)],
            out_hbm.at[c].at[pl.ds(base, ROWS_PER_TILE)],
        )

    return deg_kernel


def _make_sc_edge_scatter(ch_max, ch0, ch1):
    @functools.partial(
        pl.kernel,
        out_type=jax.ShapeDtypeStruct((NC, N_PAD, FEAT), jnp.float32),
        mesh=_mesh(),
        scratch_types=[
            pltpu.VMEM((ch_max, LANE), jnp.int32),
            pltpu.VMEM((ch_max, LANE), jnp.int32),
            pltpu.VMEM((LANE, FEAT), jnp.float32),
            pltpu.VMEM_SHARED((N_PAD, FEAT), jnp.float32),
            pltpu.SemaphoreType.DMA,
        ],
    )
    def edge_kernel(ms_hbm, src_hbm, dst_hbm, zeros_hbm, out_hbm,
                    src_v, dst_v, rows_v, acc, sem):
        c = lax.axis_index("c")
        s = lax.axis_index("s")
        wid = c * NS + s
        base = s * ROWS_PER_TILE
        nch = jnp.where(c == 0, ch0, ch1)
        pltpu.sync_copy(zeros_hbm, acc.at[pl.ds(base, ROWS_PER_TILE)])
        pltpu.sync_copy(src_hbm.at[wid], src_v)
        pltpu.sync_copy(dst_hbm.at[wid], dst_v)
        plsc.subcore_barrier()

        def body(j, carry):
            pltpu.async_copy(ms_hbm.at[src_v.at[j]], rows_v, sem).wait()
            pltpu.sync_copy(rows_v, acc.at[dst_v.at[j]], add=True)
            return carry

        lax.fori_loop(0, nch, body, 0)
        plsc.subcore_barrier()
        pltpu.sync_copy(
            acc.at[pl.ds(base, ROWS_PER_TILE)],
            out_hbm.at[c].at[pl.ds(base, ROWS_PER_TILE)],
        )

    return edge_kernel


_BLK = 1024
_GRID = N_PAD // _BLK


def _dis_from_deg(degp0, degp1):
    deg = degp0[:, 0:1] + degp1[:, 0:1] + 1.0
    return lax.rsqrt(deg)


def _tc_ms1_body(deg_ref, x_ref, w_ref, out_ref):
    dis = _dis_from_deg(deg_ref[0], deg_ref[1])
    out_ref[...] = dis * jnp.dot(x_ref[...], w_ref[...],
                                 preferred_element_type=jnp.float32)


def _tc_mid_body(deg_ref, p_ref, ms_ref, w_ref, b_ref, out_ref):
    dis = _dis_from_deg(deg_ref[0], deg_ref[1])
    agg = p_ref[0] + p_ref[1] + ms_ref[...]
    h = jax.nn.relu(dis * agg + b_ref[...])
    out_ref[...] = dis * jnp.dot(h, w_ref[...],
                                 preferred_element_type=jnp.float32)


def _tc_head_body(deg_ref, q_ref, ms_ref, b_ref, wl_ref, bl_ref, out_ref):
    dis = _dis_from_deg(deg_ref[0], deg_ref[1])
    agg = q_ref[0] + q_ref[1] + ms_ref[...]
    h = jax.nn.relu(dis * agg + b_ref[...])
    out_ref[...] = jnp.sum(h * wl_ref[...], axis=1, keepdims=True) + bl_ref[...]


def _deg_spec():
    return pl.BlockSpec((NC, _BLK, FEAT), lambda i: (0, i, 0))


def _row_spec():
    return pl.BlockSpec((_BLK, FEAT), lambda i: (i, 0))


def _part_spec():
    return pl.BlockSpec((NC, _BLK, FEAT), lambda i: (0, i, 0))


def _full_spec(shape):
    nd = len(shape)
    return pl.BlockSpec(shape, lambda i: (0,) * nd)


def _tc_ms1(degp, x, w):
    return pl.pallas_call(
        _tc_ms1_body,
        grid=(_GRID,),
        in_specs=[_deg_spec(), _row_spec(), _full_spec((FEAT, FEAT))],
        out_specs=_row_spec(),
        out_shape=jax.ShapeDtypeStruct((N_PAD, FEAT), jnp.float32),
    )(degp, x, w)


def _tc_mid(degp, part, ms, w, b):
    return pl.pallas_call(
        _tc_mid_body,
        grid=(_GRID,),
        in_specs=[_deg_spec(), _part_spec(), _row_spec(),
                  _full_spec((FEAT, FEAT)), _full_spec((1, FEAT))],
        out_specs=_row_spec(),
        out_shape=jax.ShapeDtypeStruct((N_PAD, FEAT), jnp.float32),
    )(degp, part, ms, w, b)


def _tc_head(degp, part, ms, b, wl_row, bl):
    return pl.pallas_call(
        _tc_head_body,
        grid=(_GRID,),
        in_specs=[_deg_spec(), _part_spec(), _row_spec(),
                  _full_spec((1, FEAT)), _full_spec((1, FEAT)),
                  _full_spec((1, 1))],
        out_specs=pl.BlockSpec((_BLK, 1), lambda i: (i, 0)),
        out_shape=jax.ShapeDtypeStruct((N_PAD, 1), jnp.float32),
    )(degp, part, ms, b, wl_row, bl)


@jax.jit
def _run(x, edge_index, W1, b1, W2, b2, Wl, bl):
    n, _ = x.shape
    e = edge_index.shape[1]
    src = edge_index[0].astype(jnp.int32)
    dst = edge_index[1].astype(jnp.int32)

    ch = -(-e // (NW * LANE))
    e_pad = NW * ch * LANE
    pad_idx = jnp.full((e_pad - e,), N_PAD - 1, dtype=jnp.int32)
    dst_r = jnp.concatenate([dst, pad_idx]).reshape(NW, ch, LANE)

    n_chunks = -(-e // LANE)
    ch0 = int(n_chunks * _SPLIT0) // NS
    ch1 = -(-(n_chunks - NS * ch0) // NS)
    ch_max = max(ch0, ch1)
    e0 = NS * ch0 * LANE

    def _slabs(idx):
        a = jnp.full((NC, NS, ch_max, LANE), N_PAD - 1, dtype=jnp.int32)
        p0 = idx[:e0].reshape(NS, ch0, LANE)
        p1_flat = idx[e0:]
        p1 = jnp.concatenate(
            [p1_flat,
             jnp.full((NS * ch1 * LANE - p1_flat.shape[0],), N_PAD - 1,
                      dtype=jnp.int32)]).reshape(NS, ch1, LANE)
        a = a.at[0, :, :ch0].set(p0).at[1, :, :ch1].set(p1)
        return a.reshape(NW, ch_max, LANE)

    src_s = _slabs(src)
    dst_s = _slabs(dst)

    x_pad = jnp.zeros((N_PAD, FEAT), jnp.float32).at[:n].set(x)
    ones128 = jnp.ones((LANE, FEAT), jnp.float32)
    zeros128 = jnp.zeros((ROWS_PER_TILE, FEAT), jnp.float32)

    degp = _make_sc_degree(ch)(dst_r, zeros128, ones128)

    ms1 = _tc_ms1(degp, x_pad, W1)
    p1 = _make_sc_edge_scatter(ch_max, ch0, ch1)(ms1, src_s, dst_s, zeros128)
    ms2 = _tc_mid(degp, p1, ms1, W2, b1.reshape(1, FEAT))
    p2 = _make_sc_edge_scatter(ch_max, ch0, ch1)(ms2, src_s, dst_s, zeros128)
    out = _tc_head(degp, p2, ms2, b2.reshape(1, FEAT),
                   Wl.reshape(1, FEAT), bl.reshape(1, 1))
    return out[:n, 0]


def kernel(x, edge_index, W1, b1, W2, b2, Wl, bl):
    return _run(x, edge_index, W1, b1, W2, b2, Wl, bl)
